# Initial kernel scaffold; baseline (speedup 1.0000x reference)
import jax
import jax.numpy as jnp
from jax.experimental import pallas as pl


def kernel(x_src, x_dst, edge_index):
    raise NotImplementedError("write your pallas kernel here")


# speedup vs baseline: 3.8823x; 3.8823x over previous
import functools

import jax
import jax.numpy as jnp
from jax import lax
from jax.experimental import pallas as pl
from jax.experimental.pallas import tpu as pltpu
from jax.experimental.pallas import tpu_sc as plsc

NC = 2
NS = 16
NW = NC * NS
LANES = 16
CHUNK = 80


def _edge_dot_sc(x_src, x_dst, src_idx, dst_idx):
    (e_total,) = src_idx.shape
    d = x_src.shape[1]
    assert e_total % (NW * CHUNK) == 0 and d % LANES == 0
    per_w = e_total // NW
    n_chunks = per_w // CHUNK

    mesh = plsc.VectorSubcoreMesh(
        core_axis_name="c", subcore_axis_name="s",
        num_cores=NC, num_subcores=NS)

    @functools.partial(
        pl.kernel,
        out_type=jax.ShapeDtypeStruct((e_total,), jnp.float32),
        mesh=mesh,
        compiler_params=pltpu.CompilerParams(needs_layout_passes=False),
        scratch_types=[
            pltpu.VMEM((CHUNK,), jnp.int32),
            pltpu.VMEM((CHUNK,), jnp.int32),
            pltpu.VMEM((CHUNK, d), jnp.float32),
            pltpu.VMEM((CHUNK, d), jnp.float32),
            pltpu.VMEM((CHUNK,), jnp.float32),
            pltpu.SemaphoreType.DMA,
            pltpu.SemaphoreType.DMA,
        ],
    )
    def k(xs_hbm, xd_hbm, si_hbm, di_hbm, out_hbm,
          sidx, didx, srows, drows, outv, sem_s, sem_d):
        wid = lax.axis_index("s") * NC + lax.axis_index("c")
        wbase = wid * per_w

        def chunk_body(c, _):
            base = wbase + c * CHUNK
            pltpu.sync_copy(si_hbm.at[pl.ds(base, CHUNK)], sidx)
            pltpu.sync_copy(di_hbm.at[pl.ds(base, CHUNK)], didx)
            cp_s = pltpu.async_copy(xs_hbm.at[sidx], srows, sem_s)
            cp_d = pltpu.async_copy(xd_hbm.at[didx], drows, sem_d)
            cp_s.wait()
            cp_d.wait()

            lane = lax.iota(jnp.int32, LANES)

            def group_body(g, _):
                def edge_body(l, vec):
                    e = g * LANES + l

                    def feat_body(j, acc):
                        s = srows[e, pl.ds(j * LANES, LANES)]
                        t = drows[e, pl.ds(j * LANES, LANES)]
                        return acc + s * t

                    acc = lax.fori_loop(0, d // LANES, feat_body,
                                        jnp.zeros((LANES,), jnp.float32))
                    tot = jnp.sum(acc)
                    return jnp.where(lane == l, tot, vec)

                vec = lax.fori_loop(0, LANES, edge_body,
                                    jnp.zeros((LANES,), jnp.float32))
                outv[pl.ds(g * LANES, LANES)] = vec
                return 0

            lax.fori_loop(0, CHUNK // LANES, group_body, 0)
            pltpu.sync_copy(outv, out_hbm.at[pl.ds(base, CHUNK)])
            return 0

        lax.fori_loop(0, n_chunks, chunk_body, 0)

    return k(x_src, x_dst, src_idx, dst_idx)


def kernel(x_src, x_dst, edge_index):
    e = edge_index.shape[1]
    nkey = jax.random.key(42)
    nk1, nk2 = jax.random.split(nkey)
    nsrc = jax.random.randint(nk1, (e,), 0, x_src.shape[0], dtype=jnp.int32)
    ndst = jax.random.randint(nk2, (e,), 0, x_dst.shape[0], dtype=jnp.int32)

    src_all = jnp.concatenate([edge_index[0].astype(jnp.int32), nsrc])
    dst_all = jnp.concatenate([edge_index[1].astype(jnp.int32), ndst])

    scores = _edge_dot_sc(x_src, x_dst, src_all, dst_all)
    return scores[:e], scores[e:]
